# baseline (speedup 1.0000x reference)
import functools

import jax
import jax.numpy as jnp
from jax import lax
from jax.experimental import pallas as pl
from jax.experimental.pallas import tpu as pltpu
from jax.experimental.pallas import tpu_sc as plsc

DIM = 4096
BATCH = 4096
L = 16

NC = 2
NS = 16
NW = NC * NS
ROWS_PER_W = BATCH // NW
RB = 8
NB = ROWS_PER_W // RB

_mesh = plsc.VectorSubcoreMesh(core_axis_name="c", subcore_axis_name="s")


@functools.partial(
    pl.kernel,
    out_type=jax.ShapeDtypeStruct((BATCH, DIM), jnp.float32),
    mesh=_mesh,
    scratch_types=[
        pltpu.VMEM((DIM,), jnp.int32),
        pltpu.VMEM((RB, DIM), jnp.float32),
        pltpu.VMEM((RB, DIM), jnp.float32),
    ],
    compiler_params=pltpu.CompilerParams(
        use_tc_tiling_on_sc=False, needs_layout_passes=False
    ),
)
def _permute(x_hbm, perm_hbm, out_hbm, perm_v, inb, outb):
    wid = lax.axis_index("s") * NC + lax.axis_index("c")
    row0 = wid * ROWS_PER_W
    pltpu.sync_copy(perm_hbm, perm_v)

    for b in range(NB):
        base = row0 + b * RB
        pltpu.sync_copy(x_hbm.at[pl.ds(base, RB)], inb)

        def jbody(j, carry):
            pv = perm_v[pl.ds(j * L, L)]
            for r in range(RB):
                rsel = jnp.full((L,), r, jnp.int32)
                vals = plsc.load_gather(inb, [rsel, pv])
                outb[r, pl.ds(j * L, L)] = vals
            return carry

        lax.fori_loop(0, DIM // L, jbody, 0)
        pltpu.sync_copy(outb, out_hbm.at[pl.ds(base, RB)])


def kernel(x, perm):
    return _permute(x, perm)


# speedup vs baseline: 1.7329x; 1.7329x over previous
import functools

import jax
import jax.numpy as jnp
from jax import lax
from jax.experimental import pallas as pl
from jax.experimental.pallas import tpu as pltpu
from jax.experimental.pallas import tpu_sc as plsc

DIM = 4096
BATCH = 4096
L = 16

NC = 2
NS = 16
NW = NC * NS
ROWS_PER_W = BATCH // NW
RB = 4
NB = ROWS_PER_W // RB

_mesh = plsc.VectorSubcoreMesh(core_axis_name="c", subcore_axis_name="s")


@functools.partial(
    pl.kernel,
    out_type=jax.ShapeDtypeStruct((BATCH, DIM), jnp.float32),
    mesh=_mesh,
    scratch_types=[
        pltpu.VMEM((DIM,), jnp.int32),
        pltpu.VMEM((RB, DIM), jnp.float32),
        pltpu.VMEM((RB, DIM), jnp.float32),
        pltpu.VMEM((RB, DIM), jnp.float32),
        pltpu.VMEM((RB, DIM), jnp.float32),
        pltpu.SemaphoreType.DMA,
        pltpu.SemaphoreType.DMA,
        pltpu.SemaphoreType.DMA,
        pltpu.SemaphoreType.DMA,
    ],
    compiler_params=pltpu.CompilerParams(
        use_tc_tiling_on_sc=False, needs_layout_passes=False
    ),
)
def _permute(x_hbm, perm_hbm, out_hbm, perm_v, in0, in1, out0, out1,
             si0, si1, so0, so1):
    wid = lax.axis_index("s") * NC + lax.axis_index("c")
    row0 = wid * ROWS_PER_W
    pltpu.sync_copy(perm_hbm, perm_v)

    ins = (in0, in1)
    outs = (out0, out1)
    sin = (si0, si1)
    sout = (so0, so1)

    def start_in(b):
        slot = b & 1
        return pltpu.async_copy(
            x_hbm.at[pl.ds(row0 + b * RB, RB)], ins[slot], sin[slot])

    def start_out(b):
        slot = b & 1
        return pltpu.async_copy(
            outs[slot], out_hbm.at[pl.ds(row0 + b * RB, RB)], sout[slot])

    def gather_block(src, dst):
        @plsc.parallel_loop(0, DIM, step=L, unroll=4)
        def _jloop(j):
            pv = perm_v[pl.ds(j, L)]
            for r in range(RB):
                rsel = jnp.full((L,), r, jnp.int32)
                dst[r, pl.ds(j, L)] = plsc.load_gather(src, [rsel, pv])

    in_copies = [None] * NB
    out_copies = [None] * NB
    in_copies[0] = start_in(0)
    for b in range(NB):
        slot = b & 1
        if b + 1 < NB:
            in_copies[b + 1] = start_in(b + 1)
        in_copies[b].wait()
        if b >= 2:
            out_copies[b - 2].wait()
        gather_block(ins[slot], outs[slot])
        out_copies[b] = start_out(b)
    out_copies[NB - 2].wait()
    out_copies[NB - 1].wait()


def kernel(x, perm):
    return _permute(x, perm)


# speedup vs baseline: 4.3282x; 2.4977x over previous
import functools

import jax
import jax.numpy as jnp
from jax import lax
from jax.experimental import pallas as pl
from jax.experimental.pallas import tpu as pltpu
from jax.experimental.pallas import tpu_sc as plsc

DIM = 4096
BATCH = 4096
L = 16

NC = 2
NS = 16
NW = NC * NS
ROWS_PER_W = BATCH // NW
RB = 8
NB = ROWS_PER_W // RB
HD = DIM // 2

_mesh = plsc.VectorSubcoreMesh(core_axis_name="c", subcore_axis_name="s")


@functools.partial(
    pl.kernel,
    out_type=jax.ShapeDtypeStruct((BATCH, DIM), jnp.float32),
    mesh=_mesh,
    scratch_types=[
        pltpu.VMEM((DIM,), jnp.int32),
        pltpu.VMEM((RB, DIM), jnp.float32),
        pltpu.VMEM((RB, DIM), jnp.float32),
        pltpu.VMEM((RB, HD), jnp.float32),
        pltpu.VMEM((RB, HD), jnp.float32),
        pltpu.SemaphoreType.DMA,
        pltpu.SemaphoreType.DMA,
        pltpu.SemaphoreType.DMA,
        pltpu.SemaphoreType.DMA,
    ],
    compiler_params=pltpu.CompilerParams(
        use_tc_tiling_on_sc=True, needs_layout_passes=False
    ),
)
def _permute(x_hbm, perm_hbm, out_hbm, perm_v, in0, in1, outa, outb,
             si0, si1, soa, sob):
    wid = lax.axis_index("s") * NC + lax.axis_index("c")
    row0 = wid * ROWS_PER_W
    pltpu.sync_copy(perm_hbm, perm_v)

    ins = (in0, in1)
    outs = (outa, outb)
    sin = (si0, si1)
    sout = (soa, sob)

    def start_in(b):
        slot = b & 1
        return pltpu.async_copy(
            x_hbm.at[pl.ds(row0 + b * RB, RB)], ins[slot], sin[slot])

    def start_out(b, h):
        return pltpu.async_copy(
            outs[h],
            out_hbm.at[pl.ds(row0 + b * RB, RB), pl.ds(h * HD, HD)],
            sout[h])

    def gather_half(src, dst, h):
        @plsc.parallel_loop(h * HD, (h + 1) * HD, step=L, unroll=4)
        def _jloop(j):
            pv = perm_v[pl.ds(j, L)]
            for r in range(RB):
                rsel = jnp.full((L,), r, jnp.int32)
                dst[r, pl.ds(j - h * HD, L)] = plsc.load_gather(src, [rsel, pv])

    in_copies = [None] * NB
    out_copies = [[None, None] for _ in range(NB)]
    in_copies[0] = start_in(0)
    for b in range(NB):
        slot = b & 1
        if b + 1 < NB:
            in_copies[b + 1] = start_in(b + 1)
        in_copies[b].wait()
        for h in range(2):
            if b > 0:
                out_copies[b - 1][h].wait()
            gather_half(ins[slot], outs[h], h)
            out_copies[b][h] = start_out(b, h)
    out_copies[NB - 1][0].wait()
    out_copies[NB - 1][1].wait()


def kernel(x, perm):
    return _permute(x, perm)


# speedup vs baseline: 4.8343x; 1.1169x over previous
import functools

import jax
import jax.numpy as jnp
from jax import lax
from jax.experimental import pallas as pl
from jax.experimental.pallas import tpu as pltpu
from jax.experimental.pallas import tpu_sc as plsc

DIM = 4096
BATCH = 4096
L = 16

NC = 2
NS = 16
NW = NC * NS
ROWS_PER_W = BATCH // NW
RB = 8
NB = ROWS_PER_W // RB
NP = NB // 2
HD = DIM // 2

_mesh = plsc.VectorSubcoreMesh(core_axis_name="c", subcore_axis_name="s")


@functools.partial(
    pl.kernel,
    out_type=jax.ShapeDtypeStruct((BATCH, DIM), jnp.float32),
    mesh=_mesh,
    scratch_types=[
        pltpu.VMEM((DIM,), jnp.int32),
        pltpu.VMEM((RB, DIM), jnp.float32),
        pltpu.VMEM((RB, DIM), jnp.float32),
        pltpu.VMEM((RB, HD), jnp.float32),
        pltpu.VMEM((RB, HD), jnp.float32),
        pltpu.SemaphoreType.DMA,
        pltpu.SemaphoreType.DMA,
        pltpu.SemaphoreType.DMA,
        pltpu.SemaphoreType.DMA,
    ],
    compiler_params=pltpu.CompilerParams(
        use_tc_tiling_on_sc=True, needs_layout_passes=False
    ),
)
def _permute(x_hbm, perm_hbm, out_hbm, perm_v, in0, in1, outa, outb,
             si0, si1, soa, sob):
    wid = lax.axis_index("s") * NC + lax.axis_index("c")
    row0 = wid * ROWS_PER_W
    pltpu.sync_copy(perm_hbm, perm_v)

    ins = (in0, in1)
    outs = (outa, outb)
    sin = (si0, si1)
    sout = (soa, sob)

    def start_in(b, slot):
        pltpu.make_async_copy(
            x_hbm.at[pl.ds(row0 + b * RB, RB)], ins[slot], sin[slot]).start()

    def wait_in(slot):
        pltpu.make_async_copy(
            x_hbm.at[pl.ds(0, RB)], ins[slot], sin[slot]).wait()

    def start_out(b, h):
        pltpu.make_async_copy(
            outs[h],
            out_hbm.at[pl.ds(row0 + b * RB, RB), pl.ds(h * HD, HD)],
            sout[h]).start()

    def wait_out(h):
        pltpu.make_async_copy(
            outs[h],
            out_hbm.at[pl.ds(0, RB), pl.ds(h * HD, HD)],
            sout[h]).wait()

    def gather_half(src, h):
        @plsc.parallel_loop(h * HD, (h + 1) * HD, step=L, unroll=4)
        def _jloop(j):
            pv = perm_v[pl.ds(j, L)]
            for r in range(RB):
                rsel = jnp.full((L,), r, jnp.int32)
                outs[h][r, pl.ds(j - h * HD, L)] = plsc.load_gather(
                    src, [rsel, pv])

    start_in(0, 0)
    start_in(1, 1)

    def pair_body(k, carry):
        b0 = 2 * k
        wait_in(0)
        for h in range(2):
            @pl.when(k > 0)
            def _():
                wait_out(h)
            gather_half(ins[0], h)
            start_out(b0, h)

        @pl.when(k < NP - 1)
        def _():
            start_in(b0 + 2, 0)

        wait_in(1)
        for h in range(2):
            wait_out(h)
            gather_half(ins[1], h)
            start_out(b0 + 1, h)

        @pl.when(k < NP - 1)
        def _():
            start_in(b0 + 3, 1)
        return carry

    lax.fori_loop(0, NP, pair_body, 0)
    wait_out(0)
    wait_out(1)


def kernel(x, perm):
    return _permute(x, perm)


# speedup vs baseline: 4.8992x; 1.0134x over previous
import functools

import jax
import jax.numpy as jnp
from jax import lax
from jax.experimental import pallas as pl
from jax.experimental.pallas import tpu as pltpu
from jax.experimental.pallas import tpu_sc as plsc

DIM = 4096
BATCH = 4096
L = 16

NC = 2
NS = 16
NW = NC * NS
ROWS_PER_W = BATCH // NW
RB = 8
NB = ROWS_PER_W // RB
NP = NB // 2
HD = DIM // 2

_mesh = plsc.VectorSubcoreMesh(core_axis_name="c", subcore_axis_name="s")


@functools.partial(
    pl.kernel,
    out_type=jax.ShapeDtypeStruct((BATCH, DIM), jnp.float32),
    mesh=_mesh,
    scratch_types=[
        pltpu.VMEM((DIM,), jnp.int32),
        pltpu.VMEM((RB, DIM), jnp.float32),
        pltpu.VMEM((RB, DIM), jnp.float32),
        pltpu.VMEM((RB, HD), jnp.float32),
        pltpu.VMEM((RB, HD), jnp.float32),
        pltpu.SemaphoreType.DMA,
        pltpu.SemaphoreType.DMA,
        pltpu.SemaphoreType.DMA,
        pltpu.SemaphoreType.DMA,
    ],
    compiler_params=pltpu.CompilerParams(
        use_tc_tiling_on_sc=True, needs_layout_passes=False
    ),
)
def _permute(x_hbm, perm_hbm, out_hbm, perm_v, in0, in1, outa, outb,
             si0, si1, soa, sob):
    wid = lax.axis_index("s") * NC + lax.axis_index("c")
    row0 = wid * ROWS_PER_W

    ins = (in0, in1)
    outs = (outa, outb)
    sin = (si0, si1)
    sout = (soa, sob)

    def start_in(b, slot):
        pltpu.make_async_copy(
            x_hbm.at[pl.ds(row0 + b * RB, RB)], ins[slot], sin[slot]).start()

    def wait_in(slot):
        pltpu.make_async_copy(
            x_hbm.at[pl.ds(0, RB)], ins[slot], sin[slot]).wait()

    def start_out(b, h):
        pltpu.make_async_copy(
            outs[h],
            out_hbm.at[pl.ds(row0 + b * RB, RB), pl.ds(h * HD, HD)],
            sout[h]).start()

    def wait_out(h):
        pltpu.make_async_copy(
            outs[h],
            out_hbm.at[pl.ds(0, RB), pl.ds(h * HD, HD)],
            sout[h]).wait()

    def gather_half(src, h):
        @plsc.parallel_loop(h * HD, (h + 1) * HD, step=L, unroll=2)
        def _jloop(j):
            pv = perm_v[pl.ds(j, L)]
            for r in range(RB):
                rsel = jnp.full((L,), r, jnp.int32)
                outs[h][r, pl.ds(j - h * HD, L)] = plsc.load_gather(
                    src, [rsel, pv])

    start_in(0, 0)
    start_in(1, 1)
    pltpu.sync_copy(perm_hbm, perm_v)

    def pair_body(k, carry):
        b0 = 2 * k
        wait_in(0)
        for h in range(2):
            @pl.when(k > 0)
            def _():
                wait_out(h)
            gather_half(ins[0], h)
            start_out(b0, h)

        @pl.when(k < NP - 1)
        def _():
            start_in(b0 + 2, 0)

        wait_in(1)
        for h in range(2):
            wait_out(h)
            gather_half(ins[1], h)
            start_out(b0 + 1, h)

        @pl.when(k < NP - 1)
        def _():
            start_in(b0 + 3, 1)
        return carry

    lax.fori_loop(0, NP, pair_body, 0)
    wait_out(0)
    wait_out(1)


def kernel(x, perm):
    return _permute(x, perm)
